# baseline (speedup 1.0000x reference)
import jax
import jax.numpy as jnp
from jax import lax
from jax.experimental import pallas as pl
from jax.experimental.pallas import tpu as pltpu
from jax.experimental.pallas import tpu_sc as plsc

N_NODES = 10000
N_EDGES = 160000
NC = 2
NS = 16
NW = NC * NS
LANES = 16
PER_E = N_EDGES // NW
E_BUF = 5120
UNROLL = 4
ACC_PAD = 10240


def _tc_prep_body(nf_ref, vr_ref, vi_ref):
    nf = nf_ref[...]
    rows = lax.broadcasted_iota(jnp.int32, (8, 128), 0)
    cols = lax.broadcasted_iota(jnp.int32, (8, 128), 1)
    eye = (rows == cols).astype(jnp.float32)
    vv = lax.dot_general(eye, nf, (((1,), (1,)), ((), ())),
                         preferred_element_type=jnp.float32)
    vr_ref[...] = vv[0]
    vi_ref[...] = vv[1]


def _tc_prep(nf):
    return pl.pallas_call(
        _tc_prep_body,
        out_shape=(jax.ShapeDtypeStruct((N_NODES,), jnp.float32),
                   jax.ShapeDtypeStruct((N_NODES,), jnp.float32)),
    )(nf)


def _sc_body(vr_hbm, vi_hbm, ei_hbm, p_hbm, r_hbm, x_hbm,
             out_hbm,
             vr_v, vi_v, ei_v, p_v, r_v, x_v, acc_v):
    cid = lax.axis_index("c")
    sid = lax.axis_index("s")
    wid = sid * NC + cid
    base = wid * PER_E
    ofs = pl.multiple_of(base & 127, 8)
    start_al = pl.multiple_of(base - ofs, 128)

    zf = jnp.zeros((LANES,), jnp.float32)

    pltpu.sync_copy(vr_hbm, vr_v)
    pltpu.sync_copy(vi_hbm, vi_v)
    pltpu.sync_copy(ei_hbm.at[:, pl.ds(start_al, E_BUF)], ei_v)
    pltpu.sync_copy(p_hbm.at[pl.ds(base, PER_E)], p_v.at[pl.ds(ofs, PER_E)])
    pltpu.sync_copy(r_hbm.at[pl.ds(base, PER_E)], r_v.at[pl.ds(ofs, PER_E)])
    pltpu.sync_copy(x_hbm.at[pl.ds(base, PER_E)], x_v.at[pl.ds(ofs, PER_E)])

    i16 = lax.iota(jnp.int32, LANES)
    zi = jnp.zeros((LANES,), jnp.int32)
    for row in range(2):
        for k in range(8):
            off = k * LANES
            v = ei_v[row, pl.ds(off, LANES)]
            ei_v[row, pl.ds(off, LANES)] = jnp.where(
                i16 + off < ofs, zi, v)
        for k in range(8):
            off = E_BUF - 8 * LANES + k * LANES
            v = ei_v[row, pl.ds(off, LANES)]
            ei_v[row, pl.ds(off, LANES)] = jnp.where(
                i16 + off >= ofs + jnp.int32(PER_E), zi, v)

    def zero_body(i, carry):
        acc_v[pl.ds(i * LANES, LANES)] = zf
        return carry

    lax.fori_loop(0, ACC_PAD // LANES, zero_body, 0)

    i16 = lax.iota(jnp.int32, LANES)
    z16i = jnp.zeros((LANES,), jnp.int32)
    o16i = jnp.ones((LANES,), jnp.int32)

    def do_vec(off):
        s = ei_v[0, pl.ds(off, LANES)]
        d = ei_v[1, pl.ds(off, LANES)]
        vr_s = plsc.load_gather(vr_v, [s])
        vi_s = plsc.load_gather(vi_v, [s])
        vr_d = plsc.load_gather(vr_v, [d])
        vi_d = plsc.load_gather(vi_v, [d])
        rr = r_v[pl.ds(off, LANES)] + jnp.float32(1e-6)
        xx = x_v[pl.ds(off, LANES)]
        dr = vr_s - vr_d
        di = vi_s - vi_d
        a = dr * dr + di * di
        b = rr * rr + xx * xx
        q = a / b
        ib = plsc.bitcast(q, jnp.int32)
        y = plsc.bitcast(jnp.int32(0x5F3759DF) - (ib >> 1), jnp.float32)
        half_q = jnp.float32(0.5) * q
        for _ in range(3):
            y = y * (jnp.float32(1.5) - half_q * y * y)
        w = p_v[pl.ds(off, LANES)] * q * y
        w = jnp.where(a > jnp.float32(0.0), w, jnp.float32(0.0))
        plsc.addupdate_scatter(acc_v, [s], -w)
        plsc.addupdate_scatter(acc_v, [d], w)

    def edge_body(i, carry):
        for u in range(UNROLL):
            do_vec((i * UNROLL + u) * LANES)
        return carry

    lax.fori_loop(0, E_BUF // (LANES * UNROLL), edge_body, 0)

    pltpu.sync_copy(acc_v, out_hbm.at[wid])


@jax.jit
def _sc_scatter(vr, vi, ei, p, r, x):
    mesh = plsc.VectorSubcoreMesh(
        core_axis_name="c", subcore_axis_name="s",
        num_cores=NC, num_subcores=NS)
    return pl.kernel(
        _sc_body,
        out_type=jax.ShapeDtypeStruct((NW, ACC_PAD), jnp.float32),
        mesh=mesh,
        compiler_params=pltpu.CompilerParams(needs_layout_passes=False),
        scratch_types=[
            pltpu.VMEM((N_NODES,), jnp.float32),
            pltpu.VMEM((N_NODES,), jnp.float32),
            pltpu.VMEM((2, E_BUF), jnp.int32),
            pltpu.VMEM((E_BUF,), jnp.float32),
            pltpu.VMEM((E_BUF,), jnp.float32),
            pltpu.VMEM((E_BUF,), jnp.float32),
            pltpu.VMEM((ACC_PAD,), jnp.float32),
        ],
    )(vr, vi, ei, p, r, x)


def _tc_finish_body(part_ref, out_ref):
    sums = jnp.sum(part_ref[...], axis=0)
    total = jnp.sum(sums * sums)
    out_ref[...] = jnp.reshape(total / jnp.float32(N_NODES), (1, 1))


def _tc_finish(partials):
    return pl.pallas_call(
        _tc_finish_body,
        out_shape=jax.ShapeDtypeStruct((1, 1), jnp.float32),
    )(partials)


def kernel(node_features, edge_index, edge_probs, edge_params):
    vr, vi = _tc_prep(node_features)
    partials = _sc_scatter(vr, vi, edge_index.astype(jnp.int32),
                           edge_probs, edge_params[:, 0], edge_params[:, 1])
    return _tc_finish(partials)[0, 0]


# speedup vs baseline: 1.3867x; 1.3867x over previous
import jax
import jax.numpy as jnp
from jax import lax
from jax.experimental import pallas as pl
from jax.experimental.pallas import tpu as pltpu
from jax.experimental.pallas import tpu_sc as plsc

N_NODES = 10000
N_EDGES = 160000
NC = 2
NS = 16
NW = NC * NS
LANES = 16
PER_E = N_EDGES // NW
E_BUF = 5120
UNROLL = 4
ACC_PAD = 10240


def _tc_prep_body(nf_ref, vr_ref, vi_ref):
    nf = nf_ref[...]
    rows = lax.broadcasted_iota(jnp.int32, (8, 128), 0)
    cols = lax.broadcasted_iota(jnp.int32, (8, 128), 1)
    eye = (rows == cols).astype(jnp.float32)
    vv = lax.dot_general(eye, nf, (((1,), (1,)), ((), ())),
                         preferred_element_type=jnp.float32)
    vr_ref[...] = vv[0]
    vi_ref[...] = vv[1]


def _tc_prep(nf):
    return pl.pallas_call(
        _tc_prep_body,
        out_shape=(jax.ShapeDtypeStruct((N_NODES,), jnp.float32),
                   jax.ShapeDtypeStruct((N_NODES,), jnp.float32)),
    )(nf)


def _sc_body(vr_hbm, vi_hbm, ei_hbm, p_hbm, r_hbm, x_hbm,
             out_hbm,
             vr_v, vi_v, ei_v, p_v, r_v, x_v, acc_v, sem):
    cid = lax.axis_index("c")
    sid = lax.axis_index("s")
    wid = sid * NC + cid
    base = wid * PER_E
    ofs = pl.multiple_of(base & 127, 8)
    start_al = pl.multiple_of(base - ofs, 128)

    zf = jnp.zeros((LANES,), jnp.float32)

    copies = [
        pltpu.async_copy(vr_hbm, vr_v, sem),
        pltpu.async_copy(vi_hbm, vi_v, sem),
        pltpu.async_copy(ei_hbm.at[:, pl.ds(start_al, E_BUF)], ei_v, sem),
        pltpu.async_copy(p_hbm.at[pl.ds(base, PER_E)],
                         p_v.at[pl.ds(ofs, PER_E)], sem),
        pltpu.async_copy(r_hbm.at[pl.ds(base, PER_E)],
                         r_v.at[pl.ds(ofs, PER_E)], sem),
        pltpu.async_copy(x_hbm.at[pl.ds(base, PER_E)],
                         x_v.at[pl.ds(ofs, PER_E)], sem),
    ]

    @plsc.parallel_loop(0, ACC_PAD // LANES, 1, unroll=4)
    def _(i):
        acc_v[pl.ds(i * LANES, LANES)] = zf

    for c in copies:
        c.wait()

    i16 = lax.iota(jnp.int32, LANES)
    zi = jnp.zeros((LANES,), jnp.int32)
    for row in range(2):
        for k in range(8):
            off = k * LANES
            v = ei_v[row, pl.ds(off, LANES)]
            ei_v[row, pl.ds(off, LANES)] = jnp.where(
                i16 + off < ofs, zi, v)
        for k in range(8):
            off = E_BUF - 8 * LANES + k * LANES
            v = ei_v[row, pl.ds(off, LANES)]
            ei_v[row, pl.ds(off, LANES)] = jnp.where(
                i16 + off >= ofs + jnp.int32(PER_E), zi, v)

    def do_vec(off):
        s = ei_v[0, pl.ds(off, LANES)]
        d = ei_v[1, pl.ds(off, LANES)]
        vr_s = plsc.load_gather(vr_v, [s])
        vi_s = plsc.load_gather(vi_v, [s])
        vr_d = plsc.load_gather(vr_v, [d])
        vi_d = plsc.load_gather(vi_v, [d])
        rr = r_v[pl.ds(off, LANES)] + jnp.float32(1e-6)
        xx = x_v[pl.ds(off, LANES)]
        dr = vr_s - vr_d
        di = vi_s - vi_d
        a = dr * dr + di * di
        b = rr * rr + xx * xx
        q = a / b
        ib = plsc.bitcast(q, jnp.int32)
        y = plsc.bitcast(jnp.int32(0x5F3759DF) - (ib >> 1), jnp.float32)
        half_q = jnp.float32(0.5) * q
        for _ in range(3):
            y = y * (jnp.float32(1.5) - half_q * y * y)
        w = p_v[pl.ds(off, LANES)] * q * y
        w = jnp.where(a > jnp.float32(0.0), w, jnp.float32(0.0))
        plsc.addupdate_scatter(acc_v, [s], -w)
        plsc.addupdate_scatter(acc_v, [d], w)

    @plsc.parallel_loop(0, E_BUF // LANES, 1, unroll=UNROLL)
    def _(i):
        do_vec(i * LANES)

    pltpu.sync_copy(acc_v, out_hbm.at[wid])


@jax.jit
def _sc_scatter(vr, vi, ei, p, r, x):
    mesh = plsc.VectorSubcoreMesh(
        core_axis_name="c", subcore_axis_name="s",
        num_cores=NC, num_subcores=NS)
    return pl.kernel(
        _sc_body,
        out_type=jax.ShapeDtypeStruct((NW, ACC_PAD), jnp.float32),
        mesh=mesh,
        compiler_params=pltpu.CompilerParams(needs_layout_passes=False),
        scratch_types=[
            pltpu.VMEM((N_NODES,), jnp.float32),
            pltpu.VMEM((N_NODES,), jnp.float32),
            pltpu.VMEM((2, E_BUF), jnp.int32),
            pltpu.VMEM((E_BUF,), jnp.float32),
            pltpu.VMEM((E_BUF,), jnp.float32),
            pltpu.VMEM((E_BUF,), jnp.float32),
            pltpu.VMEM((ACC_PAD,), jnp.float32),
            pltpu.SemaphoreType.DMA,
        ],
    )(vr, vi, ei, p, r, x)


def _tc_finish_body(part_ref, out_ref):
    sums = jnp.sum(part_ref[...], axis=0)
    total = jnp.sum(sums * sums)
    out_ref[...] = jnp.reshape(total / jnp.float32(N_NODES), (1, 1))


def _tc_finish(partials):
    return pl.pallas_call(
        _tc_finish_body,
        out_shape=jax.ShapeDtypeStruct((1, 1), jnp.float32),
    )(partials)


def kernel(node_features, edge_index, edge_probs, edge_params):
    vr, vi = _tc_prep(node_features)
    partials = _sc_scatter(vr, vi, edge_index.astype(jnp.int32),
                           edge_probs, edge_params[:, 0], edge_params[:, 1])
    return _tc_finish(partials)[0, 0]
